# baseline (speedup 1.0000x reference)
import functools

import jax
import jax.numpy as jnp
from jax import lax
from jax.experimental import pallas as pl
from jax.experimental.pallas import tpu as pltpu
from jax.experimental.pallas import tpu_sc as plsc

N = 10000
E = 320000
D_EDGE = 16
D_FEAT = 128
D_GLOB = 128
D_OUT = 128

NUM_CORES = 2
NUM_SUBCORES = 16
NUM_WORKERS = NUM_CORES * NUM_SUBCORES

LANE = 128
ROWS = E // LANE
ROWS_PER_W = ROWS // NUM_WORKERS
ROWS_TAIL = ROWS - ROWS_PER_W * NUM_WORKERS
DEP = 6
LAG = 2
SI = 8
EXP = 624
EXP_TAIL = N - EXP * NUM_SUBCORES


def _sc_scatter_body(ei_hbm, ea_hbm, agg_out, cnt_out,
                     idx_v, buf_v, rows_v, ones_v, zed_v, iota_v,
                     agg_sh, cnt_sh, ldsem, scsem):
    c = lax.axis_index("c")
    s = lax.axis_index("s")
    w = c * NUM_SUBCORES + s

    zrow = jnp.zeros((D_EDGE,), jnp.float32)
    orow = jnp.ones((D_EDGE,), jnp.float32)

    def _fill(i, _):
        zed_v[i, :] = zrow
        ones_v[i, :] = orow
        return 0
    lax.fori_loop(0, LANE, _fill, 0)
    iota_v[...] = lax.iota(jnp.int32, D_EDGE)

    r0 = s * EXP
    for kk in range(4):
        pltpu.sync_copy(zed_v, agg_sh.at[pl.ds(r0 + kk * LANE, LANE), :])
        pltpu.sync_copy(zed_v, cnt_sh.at[pl.ds(r0 + kk * LANE, LANE), :])
    pltpu.sync_copy(zed_v.at[pl.ds(0, EXP - 4 * LANE), :],
                    agg_sh.at[pl.ds(r0 + 4 * LANE, EXP - 4 * LANE), :])
    pltpu.sync_copy(zed_v.at[pl.ds(0, EXP - 4 * LANE), :],
                    cnt_sh.at[pl.ds(r0 + 4 * LANE, EXP - 4 * LANE), :])

    @pl.when(s == 0)
    def _zero_tail():
        t0 = EXP * NUM_SUBCORES
        pltpu.sync_copy(zed_v.at[pl.ds(0, EXP_TAIL), :],
                        agg_sh.at[pl.ds(t0, EXP_TAIL), :])
        pltpu.sync_copy(zed_v.at[pl.ds(0, EXP_TAIL), :],
                        cnt_sh.at[pl.ds(t0, EXP_TAIL), :])

    plsc.subcore_barrier()

    iota16 = lax.iota(jnp.int32, D_EDGE)

    def _fire_loads(t, i):
        pltpu.async_copy(ei_hbm.at[t, 1, :], idx_v.at[lax.rem(i, SI)], ldsem)
        pltpu.async_copy(ea_hbm.at[0, t],
                         buf_v.at[lax.rem(i, DEP), pl.ds(0, 8), :], ldsem)
        pltpu.async_copy(ea_hbm.at[1, t],
                         buf_v.at[lax.rem(i, DEP), pl.ds(8, 8), :], ldsem)

    def _drain_loads():
        pltpu.make_async_copy(ei_hbm.at[0, 1, :], idx_v.at[0], ldsem).wait()
        pltpu.make_async_copy(ea_hbm.at[0, 0],
                              buf_v.at[0, pl.ds(0, 8), :], ldsem).wait()
        pltpu.make_async_copy(ea_hbm.at[1, 0],
                              buf_v.at[0, pl.ds(8, 8), :], ldsem).wait()

    def _drain_scats():
        for _ in range(4):
            pltpu.make_async_copy(ea_hbm.at[0, 0],
                                  buf_v.at[0, pl.ds(0, 8), :], scsem).wait()

    def _transpose_tile(im, rp):
        rowsp = rows_v.at[rp]
        ebase = iota_v[...]
        for e8 in range(LANE // D_EDGE):
            ev = ebase + e8 * D_EDGE
            vs = [buf_v[im, f, pl.ds(e8 * D_EDGE, D_EDGE)]
                  for f in range(D_EDGE)]
            for f in range(D_EDGE):
                plsc.store_scatter(rowsp, [ev, jnp.full((D_EDGE,), f,
                                                        jnp.int32)], vs[f])

    base = w * ROWS_PER_W
    for j in range(DEP - 1):
        _fire_loads(base + j, j)

    def _tile(i, _):
        _drain_loads()

        @pl.when(i >= LAG)
        def _ds():
            _drain_scats()

        @pl.when(i + DEP - 1 < ROWS_PER_W)
        def _fl():
            _fire_loads(base + i + DEP - 1, i + DEP - 1)

        im = lax.rem(i, DEP)
        rp = lax.rem(i, 2)
        _transpose_tile(im, rp)
        idx = idx_v.at[lax.rem(i, SI)]
        pltpu.async_copy(rows_v.at[rp], agg_sh.at[idx], scsem, add=True)
        pltpu.async_copy(ones_v, cnt_sh.at[idx], scsem, add=True)
        return 0
    lax.fori_loop(0, ROWS_PER_W, _tile, 0)
    for _ in range(LAG):
        _drain_scats()

    @pl.when(w < ROWS_TAIL)
    def _tail():
        t = NUM_WORKERS * ROWS_PER_W + w
        pltpu.sync_copy(ei_hbm.at[t, 1, :], idx_v.at[0])
        pltpu.sync_copy(ea_hbm.at[0, t], buf_v.at[0, pl.ds(0, 8), :])
        pltpu.sync_copy(ea_hbm.at[1, t], buf_v.at[0, pl.ds(8, 8), :])
        _transpose_tile(0, 0)
        pltpu.sync_copy(rows_v.at[0], agg_sh.at[idx_v.at[0]], add=True)
        pltpu.sync_copy(ones_v, cnt_sh.at[idx_v.at[0]], add=True)

    plsc.subcore_barrier()

    pltpu.sync_copy(agg_sh.at[pl.ds(r0, EXP), :],
                    agg_out.at[c, pl.ds(r0, EXP), :])
    pltpu.sync_copy(cnt_sh.at[pl.ds(r0, EXP), :],
                    cnt_out.at[c, pl.ds(r0, EXP), :])

    @pl.when(s == 0)
    def _export_tail():
        t0 = EXP * NUM_SUBCORES
        pltpu.sync_copy(agg_sh.at[pl.ds(t0, EXP_TAIL), :],
                        agg_out.at[c, pl.ds(t0, EXP_TAIL), :])
        pltpu.sync_copy(cnt_sh.at[pl.ds(t0, EXP_TAIL), :],
                        cnt_out.at[c, pl.ds(t0, EXP_TAIL), :])


@jax.jit
def _sc_scatter(ei4, eaB):
    mesh = plsc.VectorSubcoreMesh(core_axis_name="c", subcore_axis_name="s")
    f = pl.kernel(
        _sc_scatter_body,
        mesh=mesh,
        out_type=[
            jax.ShapeDtypeStruct((NUM_CORES, N, D_EDGE), jnp.float32),
            jax.ShapeDtypeStruct((NUM_CORES, N, D_EDGE), jnp.float32),
        ],
        scratch_types=[
            pltpu.VMEM((SI, LANE), jnp.int32),
            pltpu.VMEM((DEP, D_EDGE, LANE), jnp.float32),
            pltpu.VMEM((2, LANE, D_EDGE), jnp.float32),
            pltpu.VMEM((LANE, D_EDGE), jnp.float32),
            pltpu.VMEM((LANE, D_EDGE), jnp.float32),
            pltpu.VMEM((D_EDGE,), jnp.int32),
            pltpu.VMEM_SHARED((N, D_EDGE), jnp.float32),
            pltpu.VMEM_SHARED((N, D_EDGE), jnp.float32),
            pltpu.SemaphoreType.DMA,
            pltpu.SemaphoreType.DMA,
        ],
        compiler_params=pltpu.CompilerParams(use_tc_tiling_on_sc=False,
                                             needs_layout_passes=False),
    )
    return f(ei4, eaB)


BN = 1024
BNL = BN * D_EDGE // 128
BN8 = BN // 8


def _tc_body(pagg_ref, pcnt_ref, x_ref, g_ref, w_ref, b_ref, o_ref):
    s = pagg_ref[0] + pagg_ref[1]
    c = pcnt_ref[0] + pcnt_ref[1]
    mean = (s / jnp.maximum(c, 1.0)).astype(jnp.bfloat16)
    w = w_ref[...]
    we = w[0:D_EDGE].astype(jnp.bfloat16)
    wx = w[D_EDGE:D_EDGE + D_FEAT].astype(jnp.bfloat16)
    wg = w[D_EDGE + D_FEAT:]
    gwb = (jnp.dot(g_ref[...], wg, preferred_element_type=jnp.float32)
           + b_ref[...])
    for j in range(8):
        out_j = (jnp.dot(mean[:, j * D_EDGE:(j + 1) * D_EDGE], we,
                         preferred_element_type=jnp.float32)
                 + jnp.dot(x_ref[:, j, :].astype(jnp.bfloat16), wx,
                           preferred_element_type=jnp.float32)
                 + gwb)
        o_ref[:, j, :] = out_j


@jax.jit
def _tc_combine(pagg, pcnt, x3, g2, W, b2):
    grid = (pl.cdiv(N, BN),)
    out = pl.pallas_call(
        _tc_body,
        grid=grid,
        in_specs=[
            pl.BlockSpec((NUM_CORES, BNL, 128), lambda i: (0, i, 0)),
            pl.BlockSpec((NUM_CORES, BNL, 128), lambda i: (0, i, 0)),
            pl.BlockSpec((BN8, 8, D_FEAT), lambda i: (i, 0, 0)),
            pl.BlockSpec((1, D_GLOB), lambda i: (0, 0)),
            pl.BlockSpec((D_EDGE + D_FEAT + D_GLOB, D_OUT),
                         lambda i: (0, 0)),
            pl.BlockSpec((1, D_OUT), lambda i: (0, 0)),
        ],
        out_specs=pl.BlockSpec((BN8, 8, D_OUT), lambda i: (i, 0, 0)),
        out_shape=jax.ShapeDtypeStruct((N // 8, 8, D_OUT), jnp.float32),
    )(pagg, pcnt, x3, g2, W, b2)
    return out.reshape(N, D_OUT)


def kernel(x, edge_index, edge_attr, global_attr, W, b):
    ei4 = edge_index.reshape(2, ROWS, LANE).transpose(1, 0, 2)
    eaB = edge_attr.T.reshape(2, 8, ROWS, LANE).swapaxes(1, 2)
    pagg, pcnt = _sc_scatter(ei4, eaB)
    pagg = pagg.reshape(NUM_CORES, N * D_EDGE // 128, 128)
    pcnt = pcnt.reshape(NUM_CORES, N * D_EDGE // 128, 128)
    g2 = global_attr.reshape(1, D_GLOB)
    b2 = b.reshape(1, D_OUT)
    x3 = x.reshape(N // 8, 8, D_FEAT)
    return _tc_combine(pagg, pcnt, x3, g2, W, b2)


# speedup vs baseline: 1.0196x; 1.0196x over previous
import functools

import jax
import jax.numpy as jnp
from jax import lax
from jax.experimental import pallas as pl
from jax.experimental.pallas import tpu as pltpu
from jax.experimental.pallas import tpu_sc as plsc

N = 10000
E = 320000
D_EDGE = 16
D_FEAT = 128
D_GLOB = 128
D_OUT = 128

NUM_CORES = 2
NUM_SUBCORES = 16
NUM_WORKERS = NUM_CORES * NUM_SUBCORES

LANE = 128
ROWS = E // LANE
ROWS_PER_W = ROWS // NUM_WORKERS
ROWS_TAIL = ROWS - ROWS_PER_W * NUM_WORKERS
DEP = 6
LAG = 2
SI = 8
EXP = 624
EXP_TAIL = N - EXP * NUM_SUBCORES


def _sc_scatter_body(ei_hbm, ea_hbm, agg_out, cnt_out,
                     idx_v, buf_v, rows_v, ones_v, zed_v, iota_v,
                     agg_sh, cnt_sh, ldsem, scsem):
    c = lax.axis_index("c")
    s = lax.axis_index("s")
    w = c * NUM_SUBCORES + s

    zrow = jnp.zeros((D_EDGE,), jnp.float32)
    orow = jnp.ones((D_EDGE,), jnp.float32)

    def _fill(i, _):
        zed_v[i, :] = zrow
        ones_v[i, :] = orow
        return 0
    lax.fori_loop(0, LANE, _fill, 0)
    iota_v[...] = lax.iota(jnp.int32, D_EDGE)

    r0 = s * EXP
    for kk in range(4):
        pltpu.sync_copy(zed_v, agg_sh.at[pl.ds(r0 + kk * LANE, LANE), :])
        pltpu.sync_copy(zed_v, cnt_sh.at[pl.ds(r0 + kk * LANE, LANE), :])
    pltpu.sync_copy(zed_v.at[pl.ds(0, EXP - 4 * LANE), :],
                    agg_sh.at[pl.ds(r0 + 4 * LANE, EXP - 4 * LANE), :])
    pltpu.sync_copy(zed_v.at[pl.ds(0, EXP - 4 * LANE), :],
                    cnt_sh.at[pl.ds(r0 + 4 * LANE, EXP - 4 * LANE), :])

    @pl.when(s == 0)
    def _zero_tail():
        t0 = EXP * NUM_SUBCORES
        pltpu.sync_copy(zed_v.at[pl.ds(0, EXP_TAIL), :],
                        agg_sh.at[pl.ds(t0, EXP_TAIL), :])
        pltpu.sync_copy(zed_v.at[pl.ds(0, EXP_TAIL), :],
                        cnt_sh.at[pl.ds(t0, EXP_TAIL), :])

    plsc.subcore_barrier()

    iota16 = lax.iota(jnp.int32, D_EDGE)

    def _fire_loads(t, i):
        pltpu.async_copy(ei_hbm.at[t, 1, :], idx_v.at[lax.rem(i, SI)], ldsem)
        pltpu.async_copy(ea_hbm.at[0, t],
                         buf_v.at[lax.rem(i, DEP), pl.ds(0, 8), :], ldsem)
        pltpu.async_copy(ea_hbm.at[1, t],
                         buf_v.at[lax.rem(i, DEP), pl.ds(8, 8), :], ldsem)

    def _drain_loads():
        pltpu.make_async_copy(ei_hbm.at[0, 1, :], idx_v.at[0], ldsem).wait()
        pltpu.make_async_copy(ea_hbm.at[0, 0],
                              buf_v.at[0, pl.ds(0, 8), :], ldsem).wait()
        pltpu.make_async_copy(ea_hbm.at[1, 0],
                              buf_v.at[0, pl.ds(8, 8), :], ldsem).wait()

    def _drain_scats():
        for _ in range(4):
            pltpu.make_async_copy(ea_hbm.at[0, 0],
                                  buf_v.at[0, pl.ds(0, 8), :], scsem).wait()

    def _transpose_tile(im, rp):
        rowsp = rows_v.at[rp]
        ebase = iota_v[...]
        for e8 in range(LANE // D_EDGE):
            ev = ebase + e8 * D_EDGE
            vs = [buf_v[im, f, pl.ds(e8 * D_EDGE, D_EDGE)]
                  for f in range(D_EDGE)]
            for f in range(D_EDGE):
                plsc.store_scatter(rowsp, [ev, jnp.full((D_EDGE,), f,
                                                        jnp.int32)], vs[f])

    base = w * ROWS_PER_W
    for j in range(DEP - 1):
        _fire_loads(base + j, j)

    def _tile(i, _):
        _drain_loads()

        @pl.when(i >= LAG)
        def _ds():
            _drain_scats()

        @pl.when(i + DEP - 1 < ROWS_PER_W)
        def _fl():
            _fire_loads(base + i + DEP - 1, i + DEP - 1)

        im = lax.rem(i, DEP)
        rp = lax.rem(i, 2)
        _transpose_tile(im, rp)
        idx = idx_v.at[lax.rem(i, SI)]
        pltpu.async_copy(rows_v.at[rp], agg_sh.at[idx], scsem, add=True)
        pltpu.async_copy(ones_v, cnt_sh.at[idx], scsem, add=True)
        return 0
    lax.fori_loop(0, ROWS_PER_W, _tile, 0)
    for _ in range(LAG):
        _drain_scats()

    @pl.when(w < ROWS_TAIL)
    def _tail():
        t = NUM_WORKERS * ROWS_PER_W + w
        pltpu.sync_copy(ei_hbm.at[t, 1, :], idx_v.at[0])
        pltpu.sync_copy(ea_hbm.at[0, t], buf_v.at[0, pl.ds(0, 8), :])
        pltpu.sync_copy(ea_hbm.at[1, t], buf_v.at[0, pl.ds(8, 8), :])
        _transpose_tile(0, 0)
        pltpu.sync_copy(rows_v.at[0], agg_sh.at[idx_v.at[0]], add=True)
        pltpu.sync_copy(ones_v, cnt_sh.at[idx_v.at[0]], add=True)

    plsc.subcore_barrier()

    pltpu.sync_copy(agg_sh.at[pl.ds(r0, EXP), :],
                    agg_out.at[c, pl.ds(r0, EXP), :])
    pltpu.sync_copy(cnt_sh.at[pl.ds(r0, EXP), :],
                    cnt_out.at[c, pl.ds(r0, EXP), :])

    @pl.when(s == 0)
    def _export_tail():
        t0 = EXP * NUM_SUBCORES
        pltpu.sync_copy(agg_sh.at[pl.ds(t0, EXP_TAIL), :],
                        agg_out.at[c, pl.ds(t0, EXP_TAIL), :])
        pltpu.sync_copy(cnt_sh.at[pl.ds(t0, EXP_TAIL), :],
                        cnt_out.at[c, pl.ds(t0, EXP_TAIL), :])


@jax.jit
def _sc_scatter(ei4, eaB):
    mesh = plsc.VectorSubcoreMesh(core_axis_name="c", subcore_axis_name="s")
    f = pl.kernel(
        _sc_scatter_body,
        mesh=mesh,
        out_type=[
            jax.ShapeDtypeStruct((NUM_CORES, N, D_EDGE), jnp.float32),
            jax.ShapeDtypeStruct((NUM_CORES, N, D_EDGE), jnp.float32),
        ],
        scratch_types=[
            pltpu.VMEM((SI, LANE), jnp.int32),
            pltpu.VMEM((DEP, D_EDGE, LANE), jnp.float32),
            pltpu.VMEM((2, LANE, D_EDGE), jnp.float32),
            pltpu.VMEM((LANE, D_EDGE), jnp.float32),
            pltpu.VMEM((LANE, D_EDGE), jnp.float32),
            pltpu.VMEM((D_EDGE,), jnp.int32),
            pltpu.VMEM_SHARED((N, D_EDGE), jnp.float32),
            pltpu.VMEM_SHARED((N, D_EDGE), jnp.float32),
            pltpu.SemaphoreType.DMA,
            pltpu.SemaphoreType.DMA,
        ],
        compiler_params=pltpu.CompilerParams(use_tc_tiling_on_sc=False,
                                             needs_layout_passes=False),
    )
    return f(ei4, eaB)


BN = 1024
BNL = BN * D_EDGE // 128
BN8 = BN // 8


def _tc_body(pagg_ref, pcnt_ref, x_ref, g_ref, w_ref, b_ref, o_ref):
    s = pagg_ref[0] + pagg_ref[1]
    c = pcnt_ref[0] + pcnt_ref[1]
    mean = s / jnp.maximum(c, 1.0)
    w = w_ref[...]
    we = w[0:D_EDGE]
    wx = w[D_EDGE:D_EDGE + D_FEAT]
    wg = w[D_EDGE + D_FEAT:]
    gwb = (jnp.dot(g_ref[...], wg, preferred_element_type=jnp.float32)
           + b_ref[...])
    for j in range(8):
        out_j = (jnp.dot(mean[:, j * D_EDGE:(j + 1) * D_EDGE], we,
                         preferred_element_type=jnp.float32)
                 + jnp.dot(x_ref[:, j, :], wx,
                           preferred_element_type=jnp.float32)
                 + gwb)
        o_ref[:, j, :] = out_j


@jax.jit
def _tc_combine(pagg, pcnt, x3, g2, W, b2):
    grid = (pl.cdiv(N, BN),)
    out = pl.pallas_call(
        _tc_body,
        grid=grid,
        in_specs=[
            pl.BlockSpec((NUM_CORES, BNL, 128), lambda i: (0, i, 0)),
            pl.BlockSpec((NUM_CORES, BNL, 128), lambda i: (0, i, 0)),
            pl.BlockSpec((BN8, 8, D_FEAT), lambda i: (i, 0, 0)),
            pl.BlockSpec((1, D_GLOB), lambda i: (0, 0)),
            pl.BlockSpec((D_EDGE + D_FEAT + D_GLOB, D_OUT),
                         lambda i: (0, 0)),
            pl.BlockSpec((1, D_OUT), lambda i: (0, 0)),
        ],
        out_specs=pl.BlockSpec((BN8, 8, D_OUT), lambda i: (i, 0, 0)),
        out_shape=jax.ShapeDtypeStruct((N // 8, 8, D_OUT), jnp.float32),
    )(pagg, pcnt, x3, g2, W, b2)
    return out.reshape(N, D_OUT)


def kernel(x, edge_index, edge_attr, global_attr, W, b):
    ei4 = edge_index.reshape(2, ROWS, LANE).transpose(1, 0, 2)
    eaB = edge_attr.T.reshape(2, 8, ROWS, LANE).swapaxes(1, 2)
    pagg, pcnt = _sc_scatter(ei4, eaB)
    pagg = pagg.reshape(NUM_CORES, N * D_EDGE // 128, 128)
    pcnt = pcnt.reshape(NUM_CORES, N * D_EDGE // 128, 128)
    g2 = global_attr.reshape(1, D_GLOB)
    b2 = b.reshape(1, D_OUT)
    x3 = x.reshape(N // 8, 8, D_FEAT)
    return _tc_combine(pagg, pcnt, x3, g2, W, b2)


# speedup vs baseline: 1.0536x; 1.0333x over previous
import functools

import jax
import jax.numpy as jnp
from jax import lax
from jax.experimental import pallas as pl
from jax.experimental.pallas import tpu as pltpu
from jax.experimental.pallas import tpu_sc as plsc

N = 10000
E = 320000
D_EDGE = 16
D_FEAT = 128
D_GLOB = 128
D_OUT = 128

NUM_CORES = 2
NUM_SUBCORES = 16
NUM_WORKERS = NUM_CORES * NUM_SUBCORES

LANE = 128
ROWS = E // LANE
ROWS_PER_W = ROWS // NUM_WORKERS
ROWS_TAIL = ROWS - ROWS_PER_W * NUM_WORKERS
DEP = 8
LAG = 2
SI = 12
EXP = 624
EXP_TAIL = N - EXP * NUM_SUBCORES


def _sc_scatter_body(ei_hbm, ea_hbm, agg_out, cnt_out,
                     idx_v, buf_v, rows_v, ones_v, zed_v, iota_v,
                     agg_sh, cnt_sh, ldsem, scsem):
    c = lax.axis_index("c")
    s = lax.axis_index("s")
    w = c * NUM_SUBCORES + s

    zrow = jnp.zeros((D_EDGE,), jnp.float32)
    orow = jnp.ones((D_EDGE,), jnp.float32)

    def _fill(i, _):
        zed_v[i, :] = zrow
        ones_v[i, :] = orow
        return 0
    lax.fori_loop(0, LANE, _fill, 0)
    iota_v[...] = lax.iota(jnp.int32, D_EDGE)

    r0 = s * EXP
    for kk in range(4):
        pltpu.sync_copy(zed_v, agg_sh.at[pl.ds(r0 + kk * LANE, LANE), :])
        pltpu.sync_copy(zed_v, cnt_sh.at[pl.ds(r0 + kk * LANE, LANE), :])
    pltpu.sync_copy(zed_v.at[pl.ds(0, EXP - 4 * LANE), :],
                    agg_sh.at[pl.ds(r0 + 4 * LANE, EXP - 4 * LANE), :])
    pltpu.sync_copy(zed_v.at[pl.ds(0, EXP - 4 * LANE), :],
                    cnt_sh.at[pl.ds(r0 + 4 * LANE, EXP - 4 * LANE), :])

    @pl.when(s == 0)
    def _zero_tail():
        t0 = EXP * NUM_SUBCORES
        pltpu.sync_copy(zed_v.at[pl.ds(0, EXP_TAIL), :],
                        agg_sh.at[pl.ds(t0, EXP_TAIL), :])
        pltpu.sync_copy(zed_v.at[pl.ds(0, EXP_TAIL), :],
                        cnt_sh.at[pl.ds(t0, EXP_TAIL), :])

    plsc.subcore_barrier()

    iota16 = lax.iota(jnp.int32, D_EDGE)

    def _fire_loads(t, i):
        pltpu.async_copy(ei_hbm.at[t, 1, :], idx_v.at[lax.rem(i, SI)], ldsem)
        pltpu.async_copy(ea_hbm.at[0, t],
                         buf_v.at[lax.rem(i, DEP), pl.ds(0, 8), :], ldsem)
        pltpu.async_copy(ea_hbm.at[1, t],
                         buf_v.at[lax.rem(i, DEP), pl.ds(8, 8), :], ldsem)

    def _drain_loads():
        pltpu.make_async_copy(ei_hbm.at[0, 1, :], idx_v.at[0], ldsem).wait()
        pltpu.make_async_copy(ea_hbm.at[0, 0],
                              buf_v.at[0, pl.ds(0, 8), :], ldsem).wait()
        pltpu.make_async_copy(ea_hbm.at[1, 0],
                              buf_v.at[0, pl.ds(8, 8), :], ldsem).wait()

    def _drain_scats():
        for _ in range(4):
            pltpu.make_async_copy(ea_hbm.at[0, 0],
                                  buf_v.at[0, pl.ds(0, 8), :], scsem).wait()

    def _transpose_tile(im, rp):
        rowsp = rows_v.at[rp]
        ebase = iota_v[...]
        for e8 in range(LANE // D_EDGE):
            ev = ebase + e8 * D_EDGE
            vs = [buf_v[im, f, pl.ds(e8 * D_EDGE, D_EDGE)]
                  for f in range(D_EDGE)]
            for f in range(D_EDGE):
                plsc.store_scatter(rowsp, [ev, jnp.full((D_EDGE,), f,
                                                        jnp.int32)], vs[f])

    base = w * ROWS_PER_W
    for j in range(DEP - 1):
        _fire_loads(base + j, j)

    def _tile(i, _):
        _drain_loads()

        @pl.when(i >= LAG)
        def _ds():
            _drain_scats()

        @pl.when(i + DEP - 1 < ROWS_PER_W)
        def _fl():
            _fire_loads(base + i + DEP - 1, i + DEP - 1)

        im = lax.rem(i, DEP)
        rp = lax.rem(i, 2)
        _transpose_tile(im, rp)
        idx = idx_v.at[lax.rem(i, SI)]
        pltpu.async_copy(rows_v.at[rp], agg_sh.at[idx], scsem, add=True)
        pltpu.async_copy(ones_v, cnt_sh.at[idx], scsem, add=True)
        return 0
    lax.fori_loop(0, ROWS_PER_W, _tile, 0)
    for _ in range(LAG):
        _drain_scats()

    @pl.when(w < ROWS_TAIL)
    def _tail():
        t = NUM_WORKERS * ROWS_PER_W + w
        pltpu.sync_copy(ei_hbm.at[t, 1, :], idx_v.at[0])
        pltpu.sync_copy(ea_hbm.at[0, t], buf_v.at[0, pl.ds(0, 8), :])
        pltpu.sync_copy(ea_hbm.at[1, t], buf_v.at[0, pl.ds(8, 8), :])
        _transpose_tile(0, 0)
        pltpu.sync_copy(rows_v.at[0], agg_sh.at[idx_v.at[0]], add=True)
        pltpu.sync_copy(ones_v, cnt_sh.at[idx_v.at[0]], add=True)

    plsc.subcore_barrier()

    pltpu.sync_copy(agg_sh.at[pl.ds(r0, EXP), :],
                    agg_out.at[c, pl.ds(r0, EXP), :])
    pltpu.sync_copy(cnt_sh.at[pl.ds(r0, EXP), :],
                    cnt_out.at[c, pl.ds(r0, EXP), :])

    @pl.when(s == 0)
    def _export_tail():
        t0 = EXP * NUM_SUBCORES
        pltpu.sync_copy(agg_sh.at[pl.ds(t0, EXP_TAIL), :],
                        agg_out.at[c, pl.ds(t0, EXP_TAIL), :])
        pltpu.sync_copy(cnt_sh.at[pl.ds(t0, EXP_TAIL), :],
                        cnt_out.at[c, pl.ds(t0, EXP_TAIL), :])


@jax.jit
def _sc_scatter(ei4, eaB):
    mesh = plsc.VectorSubcoreMesh(core_axis_name="c", subcore_axis_name="s")
    f = pl.kernel(
        _sc_scatter_body,
        mesh=mesh,
        out_type=[
            jax.ShapeDtypeStruct((NUM_CORES, N, D_EDGE), jnp.float32),
            jax.ShapeDtypeStruct((NUM_CORES, N, D_EDGE), jnp.float32),
        ],
        scratch_types=[
            pltpu.VMEM((SI, LANE), jnp.int32),
            pltpu.VMEM((DEP, D_EDGE, LANE), jnp.float32),
            pltpu.VMEM((2, LANE, D_EDGE), jnp.float32),
            pltpu.VMEM((LANE, D_EDGE), jnp.float32),
            pltpu.VMEM((LANE, D_EDGE), jnp.float32),
            pltpu.VMEM((D_EDGE,), jnp.int32),
            pltpu.VMEM_SHARED((N, D_EDGE), jnp.float32),
            pltpu.VMEM_SHARED((N, D_EDGE), jnp.float32),
            pltpu.SemaphoreType.DMA,
            pltpu.SemaphoreType.DMA,
        ],
        compiler_params=pltpu.CompilerParams(use_tc_tiling_on_sc=False,
                                             needs_layout_passes=False),
    )
    return f(ei4, eaB)


BN = 2048
BNL = BN * D_EDGE // 128
BN8 = BN // 8


def _tc_body(pagg_ref, pcnt_ref, x_ref, g_ref, w_ref, b_ref, o_ref):
    s = pagg_ref[0] + pagg_ref[1]
    c = pcnt_ref[0] + pcnt_ref[1]
    mean = s / jnp.maximum(c, 1.0)
    w = w_ref[...]
    we = w[0:D_EDGE]
    wx = w[D_EDGE:D_EDGE + D_FEAT]
    wg = w[D_EDGE + D_FEAT:]
    gwb = (jnp.dot(g_ref[...], wg, preferred_element_type=jnp.float32)
           + b_ref[...])
    for j in range(8):
        out_j = (jnp.dot(mean[:, j * D_EDGE:(j + 1) * D_EDGE], we,
                         preferred_element_type=jnp.float32)
                 + jnp.dot(x_ref[:, j, :], wx,
                           preferred_element_type=jnp.float32)
                 + gwb)
        o_ref[:, j, :] = out_j


@jax.jit
def _tc_combine(pagg, pcnt, x3, g2, W, b2):
    grid = (pl.cdiv(N, BN),)
    out = pl.pallas_call(
        _tc_body,
        grid=grid,
        in_specs=[
            pl.BlockSpec((NUM_CORES, BNL, 128), lambda i: (0, i, 0)),
            pl.BlockSpec((NUM_CORES, BNL, 128), lambda i: (0, i, 0)),
            pl.BlockSpec((BN8, 8, D_FEAT), lambda i: (i, 0, 0)),
            pl.BlockSpec((1, D_GLOB), lambda i: (0, 0)),
            pl.BlockSpec((D_EDGE + D_FEAT + D_GLOB, D_OUT),
                         lambda i: (0, 0)),
            pl.BlockSpec((1, D_OUT), lambda i: (0, 0)),
        ],
        out_specs=pl.BlockSpec((BN8, 8, D_OUT), lambda i: (i, 0, 0)),
        out_shape=jax.ShapeDtypeStruct((N // 8, 8, D_OUT), jnp.float32),
    )(pagg, pcnt, x3, g2, W, b2)
    return out.reshape(N, D_OUT)


def kernel(x, edge_index, edge_attr, global_attr, W, b):
    ei4 = edge_index.reshape(2, ROWS, LANE).transpose(1, 0, 2)
    eaB = edge_attr.T.reshape(2, 8, ROWS, LANE).swapaxes(1, 2)
    pagg, pcnt = _sc_scatter(ei4, eaB)
    pagg = pagg.reshape(NUM_CORES, N * D_EDGE // 128, 128)
    pcnt = pcnt.reshape(NUM_CORES, N * D_EDGE // 128, 128)
    g2 = global_attr.reshape(1, D_GLOB)
    b2 = b.reshape(1, D_OUT)
    x3 = x.reshape(N // 8, 8, D_FEAT)
    return _tc_combine(pagg, pcnt, x3, g2, W, b2)


# speedup vs baseline: 1.0706x; 1.0162x over previous
import functools

import jax
import jax.numpy as jnp
import numpy as np
from jax import lax
from jax.experimental import pallas as pl
from jax.experimental.pallas import tpu as pltpu
from jax.experimental.pallas import tpu_sc as plsc

N = 10000
E = 320000
D_EDGE = 16
D_FEAT = 128
D_GLOB = 128
D_OUT = 128

NUM_CORES = 2
NUM_SUBCORES = 16
NUM_WORKERS = NUM_CORES * NUM_SUBCORES

LANE = 128
ROWS = E // LANE
ROWS_PER_W = ROWS // NUM_WORKERS
ROWS_TAIL = ROWS - ROWS_PER_W * NUM_WORKERS
DEP = 8
LAG = 2
SI = 12
EXP = 624
EXP_TAIL = N - EXP * NUM_SUBCORES


def _sc_scatter_body(ei_hbm, ea_hbm, zo_hbm, agg_out, cnt_out,
                     idx_v, buf_v, rows_v, zo_v, zed_v, iota_v,
                     agg_sh, cnt_sh, ldsem, scsem):
    c = lax.axis_index("c")
    s = lax.axis_index("s")
    w = c * NUM_SUBCORES + s

    zrow = jnp.zeros((D_EDGE,), jnp.float32)

    def _fill(i, _):
        zed_v[i, :] = zrow
        return 0
    lax.fori_loop(0, LANE, _fill, 0)
    iota_v[...] = lax.iota(jnp.int32, D_EDGE)
    pltpu.sync_copy(zo_hbm, zo_v)
    zo_zero = zo_v.at[pl.ds(0, LANE), :]
    zo_ones = zo_v.at[pl.ds(LANE, LANE), :]

    r0 = s * EXP
    for kk in range(4):
        pltpu.sync_copy(zed_v, agg_sh.at[pl.ds(r0 + kk * LANE, LANE), :])
        pltpu.sync_copy(zo_zero, cnt_sh.at[pl.ds(r0 + kk * LANE, LANE), :])
    pltpu.sync_copy(zed_v.at[pl.ds(0, EXP - 4 * LANE), :],
                    agg_sh.at[pl.ds(r0 + 4 * LANE, EXP - 4 * LANE), :])
    pltpu.sync_copy(zo_v.at[pl.ds(0, EXP - 4 * LANE), :],
                    cnt_sh.at[pl.ds(r0 + 4 * LANE, EXP - 4 * LANE), :])

    @pl.when(s == 0)
    def _zero_tail():
        t0 = EXP * NUM_SUBCORES
        pltpu.sync_copy(zed_v.at[pl.ds(0, EXP_TAIL), :],
                        agg_sh.at[pl.ds(t0, EXP_TAIL), :])
        pltpu.sync_copy(zo_v.at[pl.ds(0, EXP_TAIL), :],
                        cnt_sh.at[pl.ds(t0, EXP_TAIL), :])

    plsc.subcore_barrier()

    iota16 = lax.iota(jnp.int32, D_EDGE)

    def _fire_loads(t, i):
        pltpu.async_copy(ei_hbm.at[t, 1, :], idx_v.at[lax.rem(i, SI)], ldsem)
        pltpu.async_copy(ea_hbm.at[0, t],
                         buf_v.at[lax.rem(i, DEP), pl.ds(0, 8), :], ldsem)
        pltpu.async_copy(ea_hbm.at[1, t],
                         buf_v.at[lax.rem(i, DEP), pl.ds(8, 8), :], ldsem)

    def _drain_loads():
        pltpu.make_async_copy(ei_hbm.at[0, 1, :], idx_v.at[0], ldsem).wait()
        pltpu.make_async_copy(ea_hbm.at[0, 0],
                              buf_v.at[0, pl.ds(0, 8), :], ldsem).wait()
        pltpu.make_async_copy(ea_hbm.at[1, 0],
                              buf_v.at[0, pl.ds(8, 8), :], ldsem).wait()

    def _drain_scats():
        for _ in range(3):
            pltpu.make_async_copy(ea_hbm.at[0, 0],
                                  buf_v.at[0, pl.ds(0, 8), :], scsem).wait()

    def _transpose_tile(im, rp):
        rowsp = rows_v.at[rp]
        ebase = iota_v[...]
        for e8 in range(LANE // D_EDGE):
            ev = ebase + e8 * D_EDGE
            vs = [buf_v[im, f, pl.ds(e8 * D_EDGE, D_EDGE)]
                  for f in range(D_EDGE)]
            for f in range(D_EDGE):
                plsc.store_scatter(rowsp, [ev, jnp.full((D_EDGE,), f,
                                                        jnp.int32)], vs[f])

    base = w * ROWS_PER_W
    for j in range(DEP - 1):
        _fire_loads(base + j, j)

    def _tile(i, _):
        _drain_loads()

        @pl.when(i >= LAG)
        def _ds():
            _drain_scats()

        @pl.when(i + DEP - 1 < ROWS_PER_W)
        def _fl():
            _fire_loads(base + i + DEP - 1, i + DEP - 1)

        im = lax.rem(i, DEP)
        rp = lax.rem(i, 2)
        _transpose_tile(im, rp)
        idx = idx_v.at[lax.rem(i, SI)]
        pltpu.async_copy(rows_v.at[rp], agg_sh.at[idx], scsem, add=True)
        pltpu.async_copy(zo_ones, cnt_sh.at[idx], scsem, add=True)
        return 0
    lax.fori_loop(0, ROWS_PER_W, _tile, 0)
    for _ in range(LAG):
        _drain_scats()

    @pl.when(w < ROWS_TAIL)
    def _tail():
        t = NUM_WORKERS * ROWS_PER_W + w
        pltpu.sync_copy(ei_hbm.at[t, 1, :], idx_v.at[0])
        pltpu.sync_copy(ea_hbm.at[0, t], buf_v.at[0, pl.ds(0, 8), :])
        pltpu.sync_copy(ea_hbm.at[1, t], buf_v.at[0, pl.ds(8, 8), :])
        _transpose_tile(0, 0)
        pltpu.sync_copy(rows_v.at[0], agg_sh.at[idx_v.at[0]], add=True)
        pltpu.sync_copy(zo_ones, cnt_sh.at[idx_v.at[0]], add=True)

    plsc.subcore_barrier()

    pltpu.sync_copy(agg_sh.at[pl.ds(r0, EXP), :],
                    agg_out.at[c, pl.ds(r0, EXP), :])
    pltpu.sync_copy(cnt_sh.at[pl.ds(r0, EXP), :],
                    cnt_out.at[c, pl.ds(r0, EXP), :])

    @pl.when(s == 0)
    def _export_tail():
        t0 = EXP * NUM_SUBCORES
        pltpu.sync_copy(agg_sh.at[pl.ds(t0, EXP_TAIL), :],
                        agg_out.at[c, pl.ds(t0, EXP_TAIL), :])
        pltpu.sync_copy(cnt_sh.at[pl.ds(t0, EXP_TAIL), :],
                        cnt_out.at[c, pl.ds(t0, EXP_TAIL), :])


D_CNT = 8


@jax.jit
def _sc_scatter(ei4, eaB, zo8):
    mesh = plsc.VectorSubcoreMesh(core_axis_name="c", subcore_axis_name="s")
    f = pl.kernel(
        _sc_scatter_body,
        mesh=mesh,
        out_type=[
            jax.ShapeDtypeStruct((NUM_CORES, N, D_EDGE), jnp.float32),
            jax.ShapeDtypeStruct((NUM_CORES, N, D_CNT), jnp.float32),
        ],
        scratch_types=[
            pltpu.VMEM((SI, LANE), jnp.int32),
            pltpu.VMEM((DEP, D_EDGE, LANE), jnp.float32),
            pltpu.VMEM((2, LANE, D_EDGE), jnp.float32),
            pltpu.VMEM((2 * LANE, D_CNT), jnp.float32),
            pltpu.VMEM((LANE, D_EDGE), jnp.float32),
            pltpu.VMEM((D_EDGE,), jnp.int32),
            pltpu.VMEM_SHARED((N, D_EDGE), jnp.float32),
            pltpu.VMEM_SHARED((N, D_CNT), jnp.float32),
            pltpu.SemaphoreType.DMA,
            pltpu.SemaphoreType.DMA,
        ],
        compiler_params=pltpu.CompilerParams(use_tc_tiling_on_sc=False,
                                             needs_layout_passes=False),
    )
    return f(ei4, eaB, zo8)


BN = 2048
BNL = BN * D_EDGE // 128
BN8 = BN // 8
BNC = BN * 8 // 128

_R_EXP = np.zeros((BNL, BNC), np.float32)
_R_EXP[np.arange(BNL), np.arange(BNL) // 2] = 1.0
_P_EVEN = np.zeros((128, 128), np.float32)
_P_ODD = np.zeros((128, 128), np.float32)
for _j in range(8):
    _P_EVEN[8 * _j, 16 * _j:16 * _j + 16] = 1.0
    _P_ODD[64 + 8 * _j, 16 * _j:16 * _j + 16] = 1.0


def _tc_body(pagg_ref, pcnt_ref, x_ref, g_ref, w_ref, b_ref,
             rr_ref, pe_ref, po_ref, o_ref):
    s = pagg_ref[0] + pagg_ref[1]
    c8 = pcnt_ref[0] + pcnt_ref[1]
    c2 = jnp.dot(rr_ref[...], c8, preferred_element_type=jnp.float32)
    ce = jnp.dot(c2, pe_ref[...], preferred_element_type=jnp.float32)
    co = jnp.dot(c2, po_ref[...], preferred_element_type=jnp.float32)
    par = lax.broadcasted_iota(jnp.int32, (BNL, 128), 0) % 2
    cexp = jnp.where(par == 0, ce, co)
    mean = s / jnp.maximum(cexp, 1.0)
    w = w_ref[...]
    we = w[0:D_EDGE]
    wx = w[D_EDGE:D_EDGE + D_FEAT]
    wg = w[D_EDGE + D_FEAT:]
    gwb = (jnp.dot(g_ref[...], wg, preferred_element_type=jnp.float32)
           + b_ref[...])
    for j in range(8):
        out_j = (jnp.dot(mean[:, j * D_EDGE:(j + 1) * D_EDGE], we,
                         preferred_element_type=jnp.float32)
                 + jnp.dot(x_ref[:, j, :], wx,
                           preferred_element_type=jnp.float32)
                 + gwb)
        o_ref[:, j, :] = out_j


@jax.jit
def _tc_combine(pagg, pcnt, x3, g2, W, b2):
    grid = (pl.cdiv(N, BN),)
    out = pl.pallas_call(
        _tc_body,
        grid=grid,
        in_specs=[
            pl.BlockSpec((NUM_CORES, BNL, 128), lambda i: (0, i, 0)),
            pl.BlockSpec((NUM_CORES, BNC, 128), lambda i: (0, i, 0)),
            pl.BlockSpec((BN8, 8, D_FEAT), lambda i: (i, 0, 0)),
            pl.BlockSpec((1, D_GLOB), lambda i: (0, 0)),
            pl.BlockSpec((D_EDGE + D_FEAT + D_GLOB, D_OUT),
                         lambda i: (0, 0)),
            pl.BlockSpec((1, D_OUT), lambda i: (0, 0)),
            pl.BlockSpec((BNL, BNC), lambda i: (0, 0)),
            pl.BlockSpec((128, 128), lambda i: (0, 0)),
            pl.BlockSpec((128, 128), lambda i: (0, 0)),
        ],
        out_specs=pl.BlockSpec((BN8, 8, D_OUT), lambda i: (i, 0, 0)),
        out_shape=jax.ShapeDtypeStruct((N // 8, 8, D_OUT), jnp.float32),
    )(pagg, pcnt, x3, g2, W, b2, jnp.asarray(_R_EXP), jnp.asarray(_P_EVEN),
      jnp.asarray(_P_ODD))
    return out.reshape(N, D_OUT)


def kernel(x, edge_index, edge_attr, global_attr, W, b):
    ei4 = edge_index.reshape(2, ROWS, LANE).transpose(1, 0, 2)
    eaB = edge_attr.T.reshape(2, 8, ROWS, LANE).swapaxes(1, 2)
    zo8 = jnp.concatenate([jnp.zeros((LANE, D_CNT), jnp.float32),
                           jnp.ones((LANE, D_CNT), jnp.float32)], axis=0)
    pagg, pcnt = _sc_scatter(ei4, eaB, zo8)
    pagg = pagg.reshape(NUM_CORES, N * D_EDGE // 128, 128)
    pcnt = pcnt.reshape(NUM_CORES, N * D_CNT // 128, 128)
    g2 = global_attr.reshape(1, D_GLOB)
    b2 = b.reshape(1, D_OUT)
    x3 = x.reshape(N // 8, 8, D_FEAT)
    return _tc_combine(pagg, pcnt, x3, g2, W, b2)


# speedup vs baseline: 1.0813x; 1.0099x over previous
import functools

import jax
import jax.numpy as jnp
import numpy as np
from jax import lax
from jax.experimental import pallas as pl
from jax.experimental.pallas import tpu as pltpu
from jax.experimental.pallas import tpu_sc as plsc

N = 10000
E = 320000
D_EDGE = 16
D_FEAT = 128
D_GLOB = 128
D_OUT = 128

NUM_CORES = 2
NUM_SUBCORES = 16
NUM_WORKERS = NUM_CORES * NUM_SUBCORES

LANE = 128
ROWS = E // LANE
ROWS_PER_W = ROWS // NUM_WORKERS
ROWS_TAIL = ROWS - ROWS_PER_W * NUM_WORKERS
DEP = 8
LAG = 2
SI = 12
EXP = 624
EXP_TAIL = N - EXP * NUM_SUBCORES


def _sc_scatter_body(ei_hbm, ea_hbm, zo_hbm, agg_out, cnt_out,
                     idx_v, buf_v, rows_v, zo_v, zed_v, iota_v,
                     agg_sh, cnt_sh, ldsem, scsem):
    c = lax.axis_index("c")
    s = lax.axis_index("s")
    w = c * NUM_SUBCORES + s

    zrow = jnp.zeros((D_EDGE,), jnp.float32)

    def _fill(i, _):
        zed_v[i, :] = zrow
        return 0
    lax.fori_loop(0, LANE, _fill, 0)
    iota_v[...] = lax.iota(jnp.int32, D_EDGE)
    pltpu.sync_copy(zo_hbm, zo_v)
    zo_zero = zo_v.at[pl.ds(0, LANE), :]
    zo_ones = zo_v.at[pl.ds(LANE, LANE), :]

    r0 = s * EXP
    zh = []
    for kk in range(4):
        zh.append(pltpu.async_copy(
            zed_v, agg_sh.at[pl.ds(r0 + kk * LANE, LANE), :], ldsem))
        zh.append(pltpu.async_copy(
            zo_zero, cnt_sh.at[pl.ds(r0 + kk * LANE, LANE), :], ldsem))
    zh.append(pltpu.async_copy(
        zed_v.at[pl.ds(0, EXP - 4 * LANE), :],
        agg_sh.at[pl.ds(r0 + 4 * LANE, EXP - 4 * LANE), :], ldsem))
    zh.append(pltpu.async_copy(
        zo_v.at[pl.ds(0, EXP - 4 * LANE), :],
        cnt_sh.at[pl.ds(r0 + 4 * LANE, EXP - 4 * LANE), :], ldsem))

    @pl.when(s == 0)
    def _zero_tail():
        t0 = EXP * NUM_SUBCORES
        pltpu.sync_copy(zed_v.at[pl.ds(0, EXP_TAIL), :],
                        agg_sh.at[pl.ds(t0, EXP_TAIL), :])
        pltpu.sync_copy(zo_v.at[pl.ds(0, EXP_TAIL), :],
                        cnt_sh.at[pl.ds(t0, EXP_TAIL), :])

    for h in zh:
        h.wait()

    plsc.subcore_barrier()

    iota16 = lax.iota(jnp.int32, D_EDGE)

    def _fire_loads(t, i):
        pltpu.async_copy(ei_hbm.at[t, 1, :], idx_v.at[lax.rem(i, SI)], ldsem)
        pltpu.async_copy(ea_hbm.at[0, t],
                         buf_v.at[lax.rem(i, DEP), pl.ds(0, 8), :], ldsem)
        pltpu.async_copy(ea_hbm.at[1, t],
                         buf_v.at[lax.rem(i, DEP), pl.ds(8, 8), :], ldsem)

    def _drain_loads():
        pltpu.make_async_copy(ei_hbm.at[0, 1, :], idx_v.at[0], ldsem).wait()
        pltpu.make_async_copy(ea_hbm.at[0, 0],
                              buf_v.at[0, pl.ds(0, 8), :], ldsem).wait()
        pltpu.make_async_copy(ea_hbm.at[1, 0],
                              buf_v.at[0, pl.ds(8, 8), :], ldsem).wait()

    def _drain_scats():
        for _ in range(3):
            pltpu.make_async_copy(ea_hbm.at[0, 0],
                                  buf_v.at[0, pl.ds(0, 8), :], scsem).wait()

    def _transpose_tile(im, rp):
        rowsp = rows_v.at[rp]
        ebase = iota_v[...]
        for e8 in range(LANE // D_EDGE):
            ev = ebase + e8 * D_EDGE
            vs = [buf_v[im, f, pl.ds(e8 * D_EDGE, D_EDGE)]
                  for f in range(D_EDGE)]
            for f in range(D_EDGE):
                plsc.store_scatter(rowsp, [ev, jnp.full((D_EDGE,), f,
                                                        jnp.int32)], vs[f])

    base = w * ROWS_PER_W
    for j in range(DEP - 1):
        _fire_loads(base + j, j)

    def _tile(i, _):
        _drain_loads()

        @pl.when(i >= LAG)
        def _ds():
            _drain_scats()

        @pl.when(i + DEP - 1 < ROWS_PER_W)
        def _fl():
            _fire_loads(base + i + DEP - 1, i + DEP - 1)

        im = lax.rem(i, DEP)
        rp = lax.rem(i, 2)
        _transpose_tile(im, rp)
        idx = idx_v.at[lax.rem(i, SI)]
        pltpu.async_copy(rows_v.at[rp], agg_sh.at[idx], scsem, add=True)
        pltpu.async_copy(zo_ones, cnt_sh.at[idx], scsem, add=True)
        return 0
    lax.fori_loop(0, ROWS_PER_W, _tile, 0)
    for _ in range(LAG):
        _drain_scats()

    @pl.when(w < ROWS_TAIL)
    def _tail():
        t = NUM_WORKERS * ROWS_PER_W + w
        pltpu.sync_copy(ei_hbm.at[t, 1, :], idx_v.at[0])
        pltpu.sync_copy(ea_hbm.at[0, t], buf_v.at[0, pl.ds(0, 8), :])
        pltpu.sync_copy(ea_hbm.at[1, t], buf_v.at[0, pl.ds(8, 8), :])
        _transpose_tile(0, 0)
        pltpu.sync_copy(rows_v.at[0], agg_sh.at[idx_v.at[0]], add=True)
        pltpu.sync_copy(zo_ones, cnt_sh.at[idx_v.at[0]], add=True)

    plsc.subcore_barrier()

    pltpu.sync_copy(agg_sh.at[pl.ds(r0, EXP), :],
                    agg_out.at[c, pl.ds(r0, EXP), :])
    pltpu.sync_copy(cnt_sh.at[pl.ds(r0, EXP), :],
                    cnt_out.at[c, pl.ds(r0, EXP), :])

    @pl.when(s == 0)
    def _export_tail():
        t0 = EXP * NUM_SUBCORES
        pltpu.sync_copy(agg_sh.at[pl.ds(t0, EXP_TAIL), :],
                        agg_out.at[c, pl.ds(t0, EXP_TAIL), :])
        pltpu.sync_copy(cnt_sh.at[pl.ds(t0, EXP_TAIL), :],
                        cnt_out.at[c, pl.ds(t0, EXP_TAIL), :])


D_CNT = 8


@jax.jit
def _sc_scatter(ei4, eaB, zo8):
    mesh = plsc.VectorSubcoreMesh(core_axis_name="c", subcore_axis_name="s")
    f = pl.kernel(
        _sc_scatter_body,
        mesh=mesh,
        out_type=[
            jax.ShapeDtypeStruct((NUM_CORES, N, D_EDGE), jnp.float32),
            jax.ShapeDtypeStruct((NUM_CORES, N, D_CNT), jnp.float32),
        ],
        scratch_types=[
            pltpu.VMEM((SI, LANE), jnp.int32),
            pltpu.VMEM((DEP, D_EDGE, LANE), jnp.float32),
            pltpu.VMEM((2, LANE, D_EDGE), jnp.float32),
            pltpu.VMEM((2 * LANE, D_CNT), jnp.float32),
            pltpu.VMEM((LANE, D_EDGE), jnp.float32),
            pltpu.VMEM((D_EDGE,), jnp.int32),
            pltpu.VMEM_SHARED((N, D_EDGE), jnp.float32),
            pltpu.VMEM_SHARED((N, D_CNT), jnp.float32),
            pltpu.SemaphoreType.DMA,
            pltpu.SemaphoreType.DMA,
        ],
        compiler_params=pltpu.CompilerParams(use_tc_tiling_on_sc=False,
                                             needs_layout_passes=False),
    )
    return f(ei4, eaB, zo8)


BN = 2048
BNL = BN * D_EDGE // 128
BN8 = BN // 8
BNC = BN * 8 // 128

_R_EXP = np.zeros((BNL, BNC), np.float32)
_R_EXP[np.arange(BNL), np.arange(BNL) // 2] = 1.0
_P_EVEN = np.zeros((128, 128), np.float32)
_P_ODD = np.zeros((128, 128), np.float32)
for _j in range(8):
    _P_EVEN[8 * _j, 16 * _j:16 * _j + 16] = 1.0
    _P_ODD[64 + 8 * _j, 16 * _j:16 * _j + 16] = 1.0


def _tc_body(pagg_ref, pcnt_ref, x_ref, g_ref, w_ref, b_ref,
             rr_ref, pe_ref, po_ref, o_ref):
    s = pagg_ref[0] + pagg_ref[1]
    c8 = pcnt_ref[0] + pcnt_ref[1]
    c2 = jnp.dot(rr_ref[...], c8, preferred_element_type=jnp.float32)
    ce = jnp.dot(c2, pe_ref[...], preferred_element_type=jnp.float32)
    co = jnp.dot(c2, po_ref[...], preferred_element_type=jnp.float32)
    par = lax.broadcasted_iota(jnp.int32, (BNL, 128), 0) % 2
    cexp = jnp.where(par == 0, ce, co)
    mean = s / jnp.maximum(cexp, 1.0)
    w = w_ref[...]
    we = w[0:D_EDGE]
    wx = w[D_EDGE:D_EDGE + D_FEAT]
    wg = w[D_EDGE + D_FEAT:]
    gwb = (jnp.dot(g_ref[...], wg, preferred_element_type=jnp.float32)
           + b_ref[...])
    for j in range(8):
        out_j = (jnp.dot(mean[:, j * D_EDGE:(j + 1) * D_EDGE], we,
                         preferred_element_type=jnp.float32)
                 + jnp.dot(x_ref[:, j, :], wx,
                           preferred_element_type=jnp.float32)
                 + gwb)
        o_ref[:, j, :] = out_j


@jax.jit
def _tc_combine(pagg, pcnt, x3, g2, W, b2):
    grid = (pl.cdiv(N, BN),)
    out = pl.pallas_call(
        _tc_body,
        grid=grid,
        in_specs=[
            pl.BlockSpec((NUM_CORES, BNL, 128), lambda i: (0, i, 0)),
            pl.BlockSpec((NUM_CORES, BNC, 128), lambda i: (0, i, 0)),
            pl.BlockSpec((BN8, 8, D_FEAT), lambda i: (i, 0, 0)),
            pl.BlockSpec((1, D_GLOB), lambda i: (0, 0)),
            pl.BlockSpec((D_EDGE + D_FEAT + D_GLOB, D_OUT),
                         lambda i: (0, 0)),
            pl.BlockSpec((1, D_OUT), lambda i: (0, 0)),
            pl.BlockSpec((BNL, BNC), lambda i: (0, 0)),
            pl.BlockSpec((128, 128), lambda i: (0, 0)),
            pl.BlockSpec((128, 128), lambda i: (0, 0)),
        ],
        out_specs=pl.BlockSpec((BN8, 8, D_OUT), lambda i: (i, 0, 0)),
        out_shape=jax.ShapeDtypeStruct((N // 8, 8, D_OUT), jnp.float32),
    )(pagg, pcnt, x3, g2, W, b2, jnp.asarray(_R_EXP), jnp.asarray(_P_EVEN),
      jnp.asarray(_P_ODD))
    return out.reshape(N, D_OUT)


def kernel(x, edge_index, edge_attr, global_attr, W, b):
    ei4 = edge_index.reshape(2, ROWS, LANE).transpose(1, 0, 2)
    eaB = edge_attr.T.reshape(2, 8, ROWS, LANE).swapaxes(1, 2)
    zo8 = jnp.concatenate([jnp.zeros((LANE, D_CNT), jnp.float32),
                           jnp.ones((LANE, D_CNT), jnp.float32)], axis=0)
    pagg, pcnt = _sc_scatter(ei4, eaB, zo8)
    pagg = pagg.reshape(NUM_CORES, N * D_EDGE // 128, 128)
    pcnt = pcnt.reshape(NUM_CORES, N * D_CNT // 128, 128)
    g2 = global_attr.reshape(1, D_GLOB)
    b2 = b.reshape(1, D_OUT)
    x3 = x.reshape(N // 8, 8, D_FEAT)
    return _tc_combine(pagg, pcnt, x3, g2, W, b2)
